# baseline (speedup 1.0000x reference)
import functools

import jax
import jax.numpy as jnp
from jax import lax
from jax.experimental import pallas as pl
from jax.experimental.pallas import tpu as pltpu

_F32 = jnp.float32
_BF = jnp.bfloat16
_G = 8


def _fold(g, b, m, v, eps=1e-5):
    s = g / jnp.sqrt(v + eps)
    return s, b - m * s


def _bd(w):
    return jnp.kron(jnp.eye(_G, dtype=w.dtype), w)


def _tile(v):
    return jnp.tile(v.reshape(-1), _G)


def _pz(a, rows, cols):
    return jnp.pad(a, ((0, rows - a.shape[0]), (0, cols - a.shape[1])))


def _pv(v, t):
    return jnp.pad(v.reshape(-1), (0, t - v.size))


def _transpose_dot(y, cout):
    eye = jnp.eye(cout, dtype=_F32)
    return lax.dot_general(eye, y, (((1,), (1,)), ((), ())),
                           preferred_element_type=_F32)


def _stem_kernel(x_ref, w00, w01, w10, w11, s_ref, b_ref, ft_ref, fb_ref,
                 *, ho, wo, rt):
    wrefs = ((0, 0, w00), (0, 1, w01), (1, 0, w10), (1, 1, w11))
    k = x_ref.shape[-1]
    c = fb_ref.shape[-1]
    for r in range(0, ho, rt):
        acc = None
        for a, b, wr in wrefs:
            xw = x_ref[0, r + a:r + a + rt, b:b + wo, :].reshape(rt * wo, k)
            t = jnp.dot(xw, wr[...], preferred_element_type=_F32)
            acc = t if acc is None else acc + t
        y = jnp.maximum(acc * s_ref[...] + b_ref[...], 0.0)
        ft_ref[0, :, r:r + rt, :] = _transpose_dot(y, c).reshape(c, rt, wo)
        fb_ref[0, r:r + rt] = y.reshape(rt, wo, c)


def _stem(s2dp, ws, scale, bias, *, ho, wo):
    n = s2dp.shape[0]
    k = s2dp.shape[-1]
    c = ws[0].shape[-1]
    rt = 16
    wspec = pl.BlockSpec((k, c), lambda i: (0, 0))
    ft, fb = pl.pallas_call(
        functools.partial(_stem_kernel, ho=ho, wo=wo, rt=rt),
        out_shape=(jax.ShapeDtypeStruct((n, c, ho, wo), _F32),
                   jax.ShapeDtypeStruct((n, ho, wo, c), _F32)),
        grid=(n,),
        in_specs=[
            pl.BlockSpec((1, ho + 1, wo + 1, k), lambda i: (i, 0, 0, 0)),
            wspec, wspec, wspec, wspec,
            pl.BlockSpec((1, c), lambda i: (0, 0)),
            pl.BlockSpec((1, c), lambda i: (0, 0)),
        ],
        out_specs=(pl.BlockSpec((1, c, ho, wo), lambda i: (i, 0, 0, 0)),
                   pl.BlockSpec((1, ho, wo, c), lambda i: (i, 0, 0, 0))),
        compiler_params=pltpu.CompilerParams(
            dimension_semantics=("parallel",)),
    )(s2dp, *ws, scale, bias)
    return ft, fb


def _dw_kernel(x_ref, w_ref, b_ref, *rest, stride, h, w, oh, ow, has_se, nc):
    if has_se:
        w1_ref, b1_ref, w2_ref, b2_ref, o_ref = rest[:5]
        xps = rest[5:]
    else:
        o_ref = rest[0]
        xps = rest[1:]
    hp, wp = h + 2, w + 2
    xv = x_ref[0].reshape(h, w, nc * 128)

    ys = []
    for kc in range(nc):
        lo = kc * 128
        xp = xps[kc]
        xp[0:1, :, :] = jnp.zeros((1, wp, 128), _F32)
        xp[hp - 1:hp, :, :] = jnp.zeros((1, wp, 128), _F32)
        xp[:, 0:1, :] = jnp.zeros((hp, 1, 128), _F32)
        xp[:, wp - 1:wp, :] = jnp.zeros((hp, 1, 128), _F32)
        xp[1:h + 1, 1:w + 1, :] = xv[:, :, lo:lo + 128]

        acc = None
        for di in range(3):
            for dj in range(3):
                t = xp[pl.ds(di, oh, stride), pl.ds(dj, ow, stride), :]
                term = t * w_ref[di * 3 + dj, :, lo:lo + 128]
                acc = term if acc is None else acc + term
        ys.append(jnp.maximum(acc + b_ref[:, lo:lo + 128], 0.0))

    if has_se:
        ms = [jnp.sum(jnp.sum(y, axis=0), axis=0, keepdims=True) for y in ys]
        m = jnp.concatenate(ms, axis=-1) * (1.0 / (oh * ow))
        hv = jnp.dot(m, w1_ref[...], preferred_element_type=_F32) + b1_ref[...]
        hv = jnp.maximum(hv, 0.0)
        g = jax.nn.sigmoid(
            jnp.dot(hv, w2_ref[...], preferred_element_type=_F32) + b2_ref[...])
        ys = [y * g[:, kc * 128:(kc + 1) * 128] for kc, y in enumerate(ys)]
    for kc, y in enumerate(ys):
        o_ref[0, :, kc * 128:(kc + 1) * 128] = (
            y.reshape(oh * ow, 128).astype(o_ref.dtype))


def _dwconv(x3d, hw, wdw, bias, se, *, stride):
    n, _, c = x3d.shape
    h, w = hw
    oh = (h - 1) // stride + 1
    ow = (w - 1) // stride + 1
    inputs = [x3d, wdw, bias.reshape(1, c)]
    specs = [
        pl.BlockSpec((1, h * w, c), lambda i: (i, 0, 0)),
        pl.BlockSpec((9, 1, c), lambda i: (0, 0, 0)),
        pl.BlockSpec((1, c), lambda i: (0, 0)),
    ]
    has_se = se is not None
    if has_se:
        w1, b1, w2, b2 = se
        rd = w1.shape[1]
        inputs += [w1, b1.reshape(1, rd), w2, b2.reshape(1, c)]
        specs += [
            pl.BlockSpec((c, rd), lambda i: (0, 0)),
            pl.BlockSpec((1, rd), lambda i: (0, 0)),
            pl.BlockSpec((rd, c), lambda i: (0, 0)),
            pl.BlockSpec((1, c), lambda i: (0, 0)),
        ]
    nc = c // 128
    return pl.pallas_call(
        functools.partial(_dw_kernel, stride=stride, h=h, w=w, oh=oh, ow=ow,
                          has_se=has_se, nc=nc),
        out_shape=jax.ShapeDtypeStruct((n, oh * ow, c), _BF),
        grid=(n,),
        in_specs=specs,
        out_specs=pl.BlockSpec((1, oh * ow, c), lambda i: (i, 0, 0)),
        scratch_shapes=[pltpu.VMEM((h + 2, w + 2, 128), _F32)] * nc,
        compiler_params=pltpu.CompilerParams(
            dimension_semantics=("parallel",)),
    )(*inputs)


def _pw_exp_kernel(x_ref, wp_ref, sp_ref, bp_ref, we_ref, se_ref, be_ref,
                   ft_ref, e_ref, *, oh, ow):
    y1 = jnp.dot(x_ref[0], wp_ref[...], preferred_element_type=_F32)
    y1 = y1 * sp_ref[...] + bp_ref[...]
    c1 = ft_ref.shape[1]
    ft_ref[0] = _transpose_dot(y1, c1).reshape(c1, oh, ow)
    y2 = jnp.dot(y1, we_ref[...], preferred_element_type=_F32)
    e_ref[0] = jnp.maximum(y2 * se_ref[...] + be_ref[...], 0.0)


def _pw_exp(x3d, hw, wp, sp, bp, we, se, be):
    n, m, k = x3d.shape
    oh, ow = hw
    c1 = wp.shape[1]
    c2 = we.shape[1]
    return pl.pallas_call(
        functools.partial(_pw_exp_kernel, oh=oh, ow=ow),
        out_shape=(jax.ShapeDtypeStruct((n, c1, oh, ow), _F32),
                   jax.ShapeDtypeStruct((n, m, c2), _F32)),
        grid=(n,),
        in_specs=[
            pl.BlockSpec((1, m, k), lambda i: (i, 0, 0)),
            pl.BlockSpec((k, c1), lambda i: (0, 0)),
            pl.BlockSpec((1, c1), lambda i: (0, 0)),
            pl.BlockSpec((1, c1), lambda i: (0, 0)),
            pl.BlockSpec((c1, c2), lambda i: (0, 0)),
            pl.BlockSpec((1, c2), lambda i: (0, 0)),
            pl.BlockSpec((1, c2), lambda i: (0, 0)),
        ],
        out_specs=(pl.BlockSpec((1, c1, oh, ow), lambda i: (i, 0, 0, 0)),
                   pl.BlockSpec((1, m, c2), lambda i: (i, 0, 0))),
        compiler_params=pltpu.CompilerParams(
            dimension_semantics=("parallel",)),
    )(x3d, wp, sp, bp, we, se, be)


def _pw_res_kernel(x_ref, wp_ref, sp_ref, bp_ref, rt_ref, ft_ref, *, oh, ow):
    y = jnp.dot(x_ref[0], wp_ref[...], preferred_element_type=_F32)
    y = y * sp_ref[...] + bp_ref[...]
    c1 = ft_ref.shape[1]
    ft_ref[0] = _transpose_dot(y, c1).reshape(c1, oh, ow) + rt_ref[0]


def _pw_res(x3d, hw, wp, sp, bp, res_t):
    n, m, k = x3d.shape
    oh, ow = hw
    c1 = wp.shape[1]
    return pl.pallas_call(
        functools.partial(_pw_res_kernel, oh=oh, ow=ow),
        out_shape=jax.ShapeDtypeStruct((n, c1, oh, ow), _F32),
        grid=(n,),
        in_specs=[
            pl.BlockSpec((1, m, k), lambda i: (i, 0, 0)),
            pl.BlockSpec((k, c1), lambda i: (0, 0)),
            pl.BlockSpec((1, c1), lambda i: (0, 0)),
            pl.BlockSpec((1, c1), lambda i: (0, 0)),
            pl.BlockSpec((1, c1, oh, ow), lambda i: (i, 0, 0, 0)),
        ],
        out_specs=pl.BlockSpec((1, c1, oh, ow), lambda i: (i, 0, 0, 0)),
        compiler_params=pltpu.CompilerParams(
            dimension_semantics=("parallel",)),
    )(x3d, wp, sp, bp, res_t)


def kernel(x, stem_w, stem_bn_g, stem_bn_b, stem_bn_m, stem_bn_v,
           s1_dw_w, s1_dw_bn_g, s1_dw_bn_b, s1_dw_bn_m, s1_dw_bn_v,
           s1_se_w1, s1_se_b1, s1_se_w2, s1_se_b2,
           s1_pw_w, s1_pw_bn_g, s1_pw_bn_b, s1_pw_bn_m, s1_pw_bn_v,
           s2_exp_w, s2_exp_bn_g, s2_exp_bn_b, s2_exp_bn_m, s2_exp_bn_v,
           s2_dw_w, s2_dw_bn_g, s2_dw_bn_b, s2_dw_bn_m, s2_dw_bn_v,
           s2_pw_w, s2_pw_bn_g, s2_pw_bn_b, s2_pw_bn_m, s2_pw_bn_v,
           s3_exp_w, s3_exp_bn_g, s3_exp_bn_b, s3_exp_bn_m, s3_exp_bn_v,
           s3_dw_w, s3_dw_bn_g, s3_dw_bn_b, s3_dw_bn_m, s3_dw_bn_v,
           s3_se_w1, s3_se_b1, s3_se_w2, s3_se_b2,
           s3_pw_w, s3_pw_bn_g, s3_pw_bn_b, s3_pw_bn_m, s3_pw_bn_v,
           s4_exp_w, s4_exp_bn_g, s4_exp_bn_b, s4_exp_bn_m, s4_exp_bn_v,
           s4_dw_w, s4_dw_bn_g, s4_dw_bn_b, s4_dw_bn_m, s4_dw_bn_v,
           s4_se_w1, s4_se_b1, s4_se_w2, s4_se_b2,
           s4_pw_w, s4_pw_bn_g, s4_pw_bn_b, s4_pw_bn_m, s4_pw_bn_v):
    n, cin, hin, _ = x.shape
    ho = hin // 2
    ng = n // _G

    s_s, s_b = _fold(stem_bn_g, stem_bn_b, stem_bn_m, stem_bn_v)
    s2d = x.astype(_BF).reshape(ng, _G, cin, ho, 2, ho, 2)
    s2d = s2d.transpose(0, 3, 5, 1, 2, 4, 6)
    s2d = s2d.reshape(ng, ho, ho, _G * 4 * cin)
    s2dp = jnp.pad(s2d, ((0, 0), (1, 0), (1, 0), (0, 0)))
    cout0 = stem_w.shape[-1]
    wpad = jnp.pad(stem_w, ((1, 0), (1, 0), (0, 0), (0, 0)))
    w4 = wpad.reshape(2, 2, 2, 2, cin, cout0).transpose(0, 2, 4, 1, 3, 5)
    w4 = w4.reshape(2, 2, 4 * cin, cout0)
    ws = tuple(_bd(w4[a, b]).astype(_BF) for a in (0, 1) for b in (0, 1))
    f0t, f0b = _stem(s2dp, ws, _tile(s_s).reshape(1, -1),
                     _tile(s_b).reshape(1, -1), ho=ho, wo=ho)

    d_s, d_b = _fold(s1_dw_bn_g, s1_dw_bn_b, s1_dw_bn_m, s1_dw_bn_v)
    wdw = jnp.tile(s1_dw_w.reshape(9, 1, -1) * d_s.reshape(1, 1, -1),
                   (1, 1, _G))
    se1 = (_bd(s1_se_w1), _tile(s1_se_b1), _bd(s1_se_w2), _tile(s1_se_b2))
    y1 = _dwconv(f0b.reshape(ng, ho * ho, -1), (ho, ho), wdw, _tile(d_b),
                 se1, stride=2)
    h1 = ho // 2
    p_s, p_b = _fold(s1_pw_bn_g, s1_pw_bn_b, s1_pw_bn_m, s1_pw_bn_v)
    e_s, e_b = _fold(s2_exp_bn_g, s2_exp_bn_b, s2_exp_bn_m, s2_exp_bn_v)
    c1 = s1_pw_w.shape[-1]
    f1t, e2 = _pw_exp(
        y1, (h1, h1), _bd(s1_pw_w.reshape(-1, c1)),
        _tile(p_s).reshape(1, -1), _tile(p_b).reshape(1, -1),
        _bd(s2_exp_w.reshape(s2_exp_w.shape[-2], -1)),
        _tile(e_s).reshape(1, -1), _tile(e_b).reshape(1, -1))

    d_s, d_b = _fold(s2_dw_bn_g, s2_dw_bn_b, s2_dw_bn_m, s2_dw_bn_v)
    wdw = jnp.tile(s2_dw_w.reshape(9, 1, -1) * d_s.reshape(1, 1, -1),
                   (1, 1, _G))
    y2 = _dwconv(e2, (h1, h1), wdw, _tile(d_b), None, stride=2)
    h2 = h1 // 2
    p_s, p_b = _fold(s2_pw_bn_g, s2_pw_bn_b, s2_pw_bn_m, s2_pw_bn_v)
    e_s, e_b = _fold(s3_exp_bn_g, s3_exp_bn_b, s3_exp_bn_m, s3_exp_bn_v)
    c2 = s2_pw_w.shape[-1]
    c3e = s3_exp_w.shape[-1]
    cp3 = ((c3e * _G + 127) // 128 * 128) // _G
    f2t, e3 = _pw_exp(
        y2, (h2, h2), _bd(s2_pw_w.reshape(-1, c2)),
        _tile(p_s).reshape(1, -1), _tile(p_b).reshape(1, -1),
        _bd(_pz(s3_exp_w.reshape(-1, c3e), s3_exp_w.shape[-2], cp3)),
        _tile(_pv(e_s, cp3)).reshape(1, -1),
        _tile(_pv(e_b, cp3)).reshape(1, -1))

    d_s, d_b = _fold(s3_dw_bn_g, s3_dw_bn_b, s3_dw_bn_m, s3_dw_bn_v)
    rd3 = s3_se_w1.shape[-1]
    wdw = jnp.tile(
        _pz((s3_dw_w.reshape(9, -1) * d_s.reshape(1, -1)), 9, cp3)
        .reshape(9, 1, cp3), (1, 1, _G))
    se3 = (_bd(_pz(s3_se_w1, cp3, rd3)), _tile(s3_se_b1),
           _bd(_pz(s3_se_w2, rd3, cp3)), _tile(_pv(s3_se_b2, cp3)))
    y3 = _dwconv(e3, (h2, h2), wdw, _tile(_pv(d_b, cp3)), se3, stride=2)
    h3 = h2 // 2
    p_s, p_b = _fold(s3_pw_bn_g, s3_pw_bn_b, s3_pw_bn_m, s3_pw_bn_v)
    e_s, e_b = _fold(s4_exp_bn_g, s4_exp_bn_b, s4_exp_bn_m, s4_exp_bn_v)
    c3 = s3_pw_w.shape[-1]
    c4e = s4_exp_w.shape[-1]
    cp4 = ((c4e * _G + 127) // 128 * 128) // _G
    f3t, e4 = _pw_exp(
        y3, (h3, h3), _bd(_pz(s3_pw_w.reshape(-1, c3), cp3, c3)),
        _tile(p_s).reshape(1, -1), _tile(p_b).reshape(1, -1),
        _bd(_pz(s4_exp_w.reshape(-1, c4e), s4_exp_w.shape[-2], cp4)),
        _tile(_pv(e_s, cp4)).reshape(1, -1),
        _tile(_pv(e_b, cp4)).reshape(1, -1))

    d_s, d_b = _fold(s4_dw_bn_g, s4_dw_bn_b, s4_dw_bn_m, s4_dw_bn_v)
    rd4 = s4_se_w1.shape[-1]
    wdw = jnp.tile(
        _pz((s4_dw_w.reshape(9, -1) * d_s.reshape(1, -1)), 9, cp4)
        .reshape(9, 1, cp4), (1, 1, _G))
    se4 = (_bd(_pz(s4_se_w1, cp4, rd4)), _tile(s4_se_b1),
           _bd(_pz(s4_se_w2, rd4, cp4)), _tile(_pv(s4_se_b2, cp4)))
    y4 = _dwconv(e4, (h3, h3), wdw, _tile(_pv(d_b, cp4)), se4, stride=1)
    p_s, p_b = _fold(s4_pw_bn_g, s4_pw_bn_b, s4_pw_bn_m, s4_pw_bn_v)
    c4 = s4_pw_w.shape[-1]
    f4t = _pw_res(
        y4, (h3, h3), _bd(_pz(s4_pw_w.reshape(-1, c4), cp4, c4)),
        _tile(p_s).reshape(1, -1), _tile(p_b).reshape(1, -1), f3t)

    return [
        f0t.reshape(n, cout0, ho, ho),
        f1t.reshape(n, c1, h1, h1),
        f2t.reshape(n, c2, h2, h2),
        f3t.reshape(n, c3, h3, h3),
        f4t.reshape(n, c4, h3, h3),
    ]


# speedup vs baseline: 5.8572x; 5.8572x over previous
import functools

import jax
import jax.numpy as jnp
from jax import lax
from jax.experimental import pallas as pl
from jax.experimental.pallas import tpu as pltpu

_F32 = jnp.float32
_BF = jnp.bfloat16
_G = 8


def _fold(g, b, m, v, eps=1e-5):
    s = g / jnp.sqrt(v + eps)
    return s, b - m * s


def _bd(w):
    return jnp.kron(jnp.eye(_G, dtype=w.dtype), w)


def _tile(v):
    return jnp.tile(v.reshape(-1), _G)


def _pz(a, rows, cols):
    return jnp.pad(a, ((0, rows - a.shape[0]), (0, cols - a.shape[1])))


def _pv(v, t):
    return jnp.pad(v.reshape(-1), (0, t - v.size))


def _transpose_dot(y, cout):
    eye = jnp.eye(cout, dtype=_F32)
    return lax.dot_general(eye, y, (((1,), (1,)), ((), ())),
                           preferred_element_type=_F32)


def _stem_kernel(x_ref, w00, w01, w10, w11, s_ref, b_ref, ft_ref, fb_ref,
                 *, ho, wo, rt):
    wrefs = ((0, 0, w00), (0, 1, w01), (1, 0, w10), (1, 1, w11))
    k = x_ref.shape[-1]
    c = fb_ref.shape[-1]
    for r in range(0, ho, rt):
        acc = None
        for a, b, wr in wrefs:
            xw = x_ref[0, r + a:r + a + rt, b:b + wo, :].reshape(rt * wo, k)
            t = jnp.dot(xw, wr[...], preferred_element_type=_F32)
            acc = t if acc is None else acc + t
        y = jnp.maximum(acc * s_ref[...] + b_ref[...], 0.0)
        ft_ref[0, :, r:r + rt, :] = _transpose_dot(y, c).reshape(c, rt, wo)
        fb_ref[0, r:r + rt] = y.reshape(rt, wo, c)


def _stem(s2dp, ws, scale, bias, *, ho, wo):
    n = s2dp.shape[0]
    k = s2dp.shape[-1]
    c = ws[0].shape[-1]
    rt = 16
    wspec = pl.BlockSpec((k, c), lambda i: (0, 0))
    ft, fb = pl.pallas_call(
        functools.partial(_stem_kernel, ho=ho, wo=wo, rt=rt),
        out_shape=(jax.ShapeDtypeStruct((n, c, ho, wo), _F32),
                   jax.ShapeDtypeStruct((n, ho, wo, c), _F32)),
        grid=(n,),
        in_specs=[
            pl.BlockSpec((1, ho + 1, wo + 1, k), lambda i: (i, 0, 0, 0)),
            wspec, wspec, wspec, wspec,
            pl.BlockSpec((1, c), lambda i: (0, 0)),
            pl.BlockSpec((1, c), lambda i: (0, 0)),
        ],
        out_specs=(pl.BlockSpec((1, c, ho, wo), lambda i: (i, 0, 0, 0)),
                   pl.BlockSpec((1, ho, wo, c), lambda i: (i, 0, 0, 0))),
        compiler_params=pltpu.CompilerParams(
            dimension_semantics=("parallel",)),
    )(s2dp, *ws, scale, bias)
    return ft, fb


def _dw_kernel(x_ref, w_ref, b_ref, *rest, stride, h, w, oh, ow, has_se, nc):
    if has_se:
        w1_ref, b1_ref, w2_ref, b2_ref, o_ref = rest[:5]
        xps = rest[5:]
    else:
        o_ref = rest[0]
        xps = rest[1:]
    hp, wp = h + 2, w + 2
    xv = x_ref[0].reshape(h, w, nc * 128)

    ys = []
    for kc in range(nc):
        lo = kc * 128
        xp = xps[kc]
        xp[0:1, :, :] = jnp.zeros((1, wp, 128), _F32)
        xp[hp - 1:hp, :, :] = jnp.zeros((1, wp, 128), _F32)
        xp[:, 0:1, :] = jnp.zeros((hp, 1, 128), _F32)
        xp[:, wp - 1:wp, :] = jnp.zeros((hp, 1, 128), _F32)
        xp[1:h + 1, 1:w + 1, :] = xv[:, :, lo:lo + 128]

        acc = None
        for di in range(3):
            for dj in range(3):
                t = xp[pl.ds(di, oh, stride), pl.ds(dj, ow, stride), :]
                term = t * w_ref[di * 3 + dj, :, lo:lo + 128]
                acc = term if acc is None else acc + term
        ys.append(jnp.maximum(acc + b_ref[:, lo:lo + 128], 0.0))

    if has_se:
        ms = [jnp.sum(jnp.sum(y, axis=0), axis=0, keepdims=True) for y in ys]
        m = jnp.concatenate(ms, axis=-1) * (1.0 / (oh * ow))
        hv = jnp.dot(m, w1_ref[...], preferred_element_type=_F32) + b1_ref[...]
        hv = jnp.maximum(hv, 0.0)
        g = jax.nn.sigmoid(
            jnp.dot(hv, w2_ref[...], preferred_element_type=_F32) + b2_ref[...])
        ys = [y * g[:, kc * 128:(kc + 1) * 128] for kc, y in enumerate(ys)]
    for kc, y in enumerate(ys):
        o_ref[0, :, kc * 128:(kc + 1) * 128] = (
            y.reshape(oh * ow, 128).astype(o_ref.dtype))


def _dwconv(x3d, hw, wdw, bias, se, *, stride):
    n, _, c = x3d.shape
    h, w = hw
    oh = (h - 1) // stride + 1
    ow = (w - 1) // stride + 1
    inputs = [x3d, wdw, bias.reshape(1, c)]
    specs = [
        pl.BlockSpec((1, h * w, c), lambda i: (i, 0, 0)),
        pl.BlockSpec((9, 1, c), lambda i: (0, 0, 0)),
        pl.BlockSpec((1, c), lambda i: (0, 0)),
    ]
    has_se = se is not None
    if has_se:
        w1, b1, w2, b2 = se
        rd = w1.shape[1]
        inputs += [w1, b1.reshape(1, rd), w2, b2.reshape(1, c)]
        specs += [
            pl.BlockSpec((c, rd), lambda i: (0, 0)),
            pl.BlockSpec((1, rd), lambda i: (0, 0)),
            pl.BlockSpec((rd, c), lambda i: (0, 0)),
            pl.BlockSpec((1, c), lambda i: (0, 0)),
        ]
    nc = c // 128
    return pl.pallas_call(
        functools.partial(_dw_kernel, stride=stride, h=h, w=w, oh=oh, ow=ow,
                          has_se=has_se, nc=nc),
        out_shape=jax.ShapeDtypeStruct((n, oh * ow, c), _BF),
        grid=(n,),
        in_specs=specs,
        out_specs=pl.BlockSpec((1, oh * ow, c), lambda i: (i, 0, 0)),
        scratch_shapes=[pltpu.VMEM((h + 2, w + 2, 128), _F32)] * nc,
        compiler_params=pltpu.CompilerParams(
            dimension_semantics=("parallel",)),
    )(*inputs)


def _pw_exp_kernel(x_ref, wp_ref, sp_ref, bp_ref, we_ref, se_ref, be_ref,
                   ft_ref, e_ref, *, oh, ow):
    y1 = jnp.dot(x_ref[0], wp_ref[...], preferred_element_type=_F32)
    y1 = y1 * sp_ref[...] + bp_ref[...]
    c1 = ft_ref.shape[1]
    ft_ref[0] = _transpose_dot(y1, c1).reshape(c1, oh, ow)
    y2 = jnp.dot(y1, we_ref[...], preferred_element_type=_F32)
    e_ref[0] = jnp.maximum(y2 * se_ref[...] + be_ref[...], 0.0)


def _pw_exp(x3d, hw, wp, sp, bp, we, se, be):
    n, m, k = x3d.shape
    oh, ow = hw
    c1 = wp.shape[1]
    c2 = we.shape[1]
    return pl.pallas_call(
        functools.partial(_pw_exp_kernel, oh=oh, ow=ow),
        out_shape=(jax.ShapeDtypeStruct((n, c1, oh, ow), _F32),
                   jax.ShapeDtypeStruct((n, m, c2), _F32)),
        grid=(n,),
        in_specs=[
            pl.BlockSpec((1, m, k), lambda i: (i, 0, 0)),
            pl.BlockSpec((k, c1), lambda i: (0, 0)),
            pl.BlockSpec((1, c1), lambda i: (0, 0)),
            pl.BlockSpec((1, c1), lambda i: (0, 0)),
            pl.BlockSpec((c1, c2), lambda i: (0, 0)),
            pl.BlockSpec((1, c2), lambda i: (0, 0)),
            pl.BlockSpec((1, c2), lambda i: (0, 0)),
        ],
        out_specs=(pl.BlockSpec((1, c1, oh, ow), lambda i: (i, 0, 0, 0)),
                   pl.BlockSpec((1, m, c2), lambda i: (i, 0, 0))),
        compiler_params=pltpu.CompilerParams(
            dimension_semantics=("parallel",)),
    )(x3d, wp, sp, bp, we, se, be)


def _pw_res_kernel(x_ref, wp_ref, sp_ref, bp_ref, rt_ref, ft_ref, *, oh, ow):
    y = jnp.dot(x_ref[0], wp_ref[...], preferred_element_type=_F32)
    y = y * sp_ref[...] + bp_ref[...]
    c1 = ft_ref.shape[1]
    ft_ref[0] = _transpose_dot(y, c1).reshape(c1, oh, ow) + rt_ref[0]


def _pw_res(x3d, hw, wp, sp, bp, res_t):
    n, m, k = x3d.shape
    oh, ow = hw
    c1 = wp.shape[1]
    return pl.pallas_call(
        functools.partial(_pw_res_kernel, oh=oh, ow=ow),
        out_shape=jax.ShapeDtypeStruct((n, c1, oh, ow), _F32),
        grid=(n,),
        in_specs=[
            pl.BlockSpec((1, m, k), lambda i: (i, 0, 0)),
            pl.BlockSpec((k, c1), lambda i: (0, 0)),
            pl.BlockSpec((1, c1), lambda i: (0, 0)),
            pl.BlockSpec((1, c1), lambda i: (0, 0)),
            pl.BlockSpec((1, c1, oh, ow), lambda i: (i, 0, 0, 0)),
        ],
        out_specs=pl.BlockSpec((1, c1, oh, ow), lambda i: (i, 0, 0, 0)),
        compiler_params=pltpu.CompilerParams(
            dimension_semantics=("parallel",)),
    )(x3d, wp, sp, bp, res_t)


def kernel(x, stem_w, stem_bn_g, stem_bn_b, stem_bn_m, stem_bn_v,
           s1_dw_w, s1_dw_bn_g, s1_dw_bn_b, s1_dw_bn_m, s1_dw_bn_v,
           s1_se_w1, s1_se_b1, s1_se_w2, s1_se_b2,
           s1_pw_w, s1_pw_bn_g, s1_pw_bn_b, s1_pw_bn_m, s1_pw_bn_v,
           s2_exp_w, s2_exp_bn_g, s2_exp_bn_b, s2_exp_bn_m, s2_exp_bn_v,
           s2_dw_w, s2_dw_bn_g, s2_dw_bn_b, s2_dw_bn_m, s2_dw_bn_v,
           s2_pw_w, s2_pw_bn_g, s2_pw_bn_b, s2_pw_bn_m, s2_pw_bn_v,
           s3_exp_w, s3_exp_bn_g, s3_exp_bn_b, s3_exp_bn_m, s3_exp_bn_v,
           s3_dw_w, s3_dw_bn_g, s3_dw_bn_b, s3_dw_bn_m, s3_dw_bn_v,
           s3_se_w1, s3_se_b1, s3_se_w2, s3_se_b2,
           s3_pw_w, s3_pw_bn_g, s3_pw_bn_b, s3_pw_bn_m, s3_pw_bn_v,
           s4_exp_w, s4_exp_bn_g, s4_exp_bn_b, s4_exp_bn_m, s4_exp_bn_v,
           s4_dw_w, s4_dw_bn_g, s4_dw_bn_b, s4_dw_bn_m, s4_dw_bn_v,
           s4_se_w1, s4_se_b1, s4_se_w2, s4_se_b2,
           s4_pw_w, s4_pw_bn_g, s4_pw_bn_b, s4_pw_bn_m, s4_pw_bn_v):
    n, cin, hin, _ = x.shape
    ho = hin // 2
    ng = n // _G

    s_s, s_b = _fold(stem_bn_g, stem_bn_b, stem_bn_m, stem_bn_v)
    s2d = x.reshape(ng, _G, cin, ho, 2, ho, 2)
    s2d = s2d.transpose(0, 3, 5, 1, 2, 4, 6)
    s2d = s2d.reshape(ng, ho, ho, _G * 4 * cin)
    s2dp = jnp.pad(s2d, ((0, 0), (1, 0), (1, 0), (0, 0)))
    cout0 = stem_w.shape[-1]
    wpad = jnp.pad(stem_w, ((1, 0), (1, 0), (0, 0), (0, 0)))
    w4 = wpad.reshape(2, 2, 2, 2, cin, cout0).transpose(0, 2, 4, 1, 3, 5)
    w4 = w4.reshape(2, 2, 4 * cin, cout0)
    ws = tuple(_bd(w4[a, b]) for a in (0, 1) for b in (0, 1))
    f0t, f0b = _stem(s2dp, ws, _tile(s_s).reshape(1, -1),
                     _tile(s_b).reshape(1, -1), ho=ho, wo=ho)

    d_s, d_b = _fold(s1_dw_bn_g, s1_dw_bn_b, s1_dw_bn_m, s1_dw_bn_v)
    wdw = jnp.tile(s1_dw_w.reshape(9, 1, -1) * d_s.reshape(1, 1, -1),
                   (1, 1, _G))
    se1 = (_bd(s1_se_w1), _tile(s1_se_b1), _bd(s1_se_w2), _tile(s1_se_b2))
    y1 = _dwconv(f0b.reshape(ng, ho * ho, -1), (ho, ho), wdw, _tile(d_b),
                 se1, stride=2)
    h1 = ho // 2
    p_s, p_b = _fold(s1_pw_bn_g, s1_pw_bn_b, s1_pw_bn_m, s1_pw_bn_v)
    e_s, e_b = _fold(s2_exp_bn_g, s2_exp_bn_b, s2_exp_bn_m, s2_exp_bn_v)
    c1 = s1_pw_w.shape[-1]
    f1t, e2 = _pw_exp(
        y1, (h1, h1), _bd(s1_pw_w.reshape(-1, c1)),
        _tile(p_s).reshape(1, -1), _tile(p_b).reshape(1, -1),
        _bd(s2_exp_w.reshape(s2_exp_w.shape[-2], -1)),
        _tile(e_s).reshape(1, -1), _tile(e_b).reshape(1, -1))

    d_s, d_b = _fold(s2_dw_bn_g, s2_dw_bn_b, s2_dw_bn_m, s2_dw_bn_v)
    wdw = jnp.tile(s2_dw_w.reshape(9, 1, -1) * d_s.reshape(1, 1, -1),
                   (1, 1, _G))
    y2 = _dwconv(e2, (h1, h1), wdw, _tile(d_b), None, stride=2)
    h2 = h1 // 2
    p_s, p_b = _fold(s2_pw_bn_g, s2_pw_bn_b, s2_pw_bn_m, s2_pw_bn_v)
    e_s, e_b = _fold(s3_exp_bn_g, s3_exp_bn_b, s3_exp_bn_m, s3_exp_bn_v)
    c2 = s2_pw_w.shape[-1]
    c3e = s3_exp_w.shape[-1]
    cp3 = ((c3e * _G + 127) // 128 * 128) // _G
    f2t, e3 = _pw_exp(
        y2, (h2, h2), _bd(s2_pw_w.reshape(-1, c2)),
        _tile(p_s).reshape(1, -1), _tile(p_b).reshape(1, -1),
        _bd(_pz(s3_exp_w.reshape(-1, c3e), s3_exp_w.shape[-2], cp3)),
        _tile(_pv(e_s, cp3)).reshape(1, -1),
        _tile(_pv(e_b, cp3)).reshape(1, -1))

    d_s, d_b = _fold(s3_dw_bn_g, s3_dw_bn_b, s3_dw_bn_m, s3_dw_bn_v)
    rd3 = s3_se_w1.shape[-1]
    wdw = jnp.tile(
        _pz((s3_dw_w.reshape(9, -1) * d_s.reshape(1, -1)), 9, cp3)
        .reshape(9, 1, cp3), (1, 1, _G))
    se3 = (_bd(_pz(s3_se_w1, cp3, rd3)), _tile(s3_se_b1),
           _bd(_pz(s3_se_w2, rd3, cp3)), _tile(_pv(s3_se_b2, cp3)))
    y3 = _dwconv(e3, (h2, h2), wdw, _tile(_pv(d_b, cp3)), se3, stride=2)
    h3 = h2 // 2
    p_s, p_b = _fold(s3_pw_bn_g, s3_pw_bn_b, s3_pw_bn_m, s3_pw_bn_v)
    e_s, e_b = _fold(s4_exp_bn_g, s4_exp_bn_b, s4_exp_bn_m, s4_exp_bn_v)
    c3 = s3_pw_w.shape[-1]
    c4e = s4_exp_w.shape[-1]
    cp4 = ((c4e * _G + 127) // 128 * 128) // _G
    f3t, e4 = _pw_exp(
        y3, (h3, h3), _bd(_pz(s3_pw_w.reshape(-1, c3), cp3, c3)),
        _tile(p_s).reshape(1, -1), _tile(p_b).reshape(1, -1),
        _bd(_pz(s4_exp_w.reshape(-1, c4e), s4_exp_w.shape[-2], cp4)),
        _tile(_pv(e_s, cp4)).reshape(1, -1),
        _tile(_pv(e_b, cp4)).reshape(1, -1))

    d_s, d_b = _fold(s4_dw_bn_g, s4_dw_bn_b, s4_dw_bn_m, s4_dw_bn_v)
    rd4 = s4_se_w1.shape[-1]
    wdw = jnp.tile(
        _pz((s4_dw_w.reshape(9, -1) * d_s.reshape(1, -1)), 9, cp4)
        .reshape(9, 1, cp4), (1, 1, _G))
    se4 = (_bd(_pz(s4_se_w1, cp4, rd4)), _tile(s4_se_b1),
           _bd(_pz(s4_se_w2, rd4, cp4)), _tile(_pv(s4_se_b2, cp4)))
    y4 = _dwconv(e4, (h3, h3), wdw, _tile(_pv(d_b, cp4)), se4, stride=1)
    p_s, p_b = _fold(s4_pw_bn_g, s4_pw_bn_b, s4_pw_bn_m, s4_pw_bn_v)
    c4 = s4_pw_w.shape[-1]
    f4t = _pw_res(
        y4, (h3, h3), _bd(_pz(s4_pw_w.reshape(-1, c4), cp4, c4)),
        _tile(p_s).reshape(1, -1), _tile(p_b).reshape(1, -1), f3t)

    return [
        f0t.reshape(n, cout0, ho, ho),
        f1t.reshape(n, c1, h1, h1),
        f2t.reshape(n, c2, h2, h2),
        f3t.reshape(n, c3, h3, h3),
        f4t.reshape(n, c4, h3, h3),
    ]
